# Initial kernel scaffold; baseline (speedup 1.0000x reference)
import jax
import jax.numpy as jnp
from jax.experimental import pallas as pl


def kernel(dev_sh, dev_spv, dev_spr, dev_level, sh_W1, sh_b1, sh_W2, sh_b2, c1_W, c1_b, cmu_W, cmu_b, cls_W, cls_b, spv_W1, spv_b1, spv_W2, spv_b2, spr_W1, spr_b1, spr_W2, spr_b2, eps):
    raise NotImplementedError("write your pallas kernel here")


# speedup vs baseline: 2.7642x; 2.7642x over previous
import numpy as np
import jax
import jax.numpy as jnp
from jax.experimental import pallas as pl
from jax.experimental.pallas import tpu as pltpu


def _build_selector():
    S = np.zeros((256, 16), dtype=np.float32)
    for m in range(4):
        for n in range(4):
            for k in range(16):
                S[m * 64 + n * 16 + k, n * 4 + m] = 1.0
    return S


_S_NP = _build_selector()


def _fused_kernel(spv_ref, sh_ref, level_ref, spr_ref, eps_ref,
                  Wa_ref, ba_ref, Wb_ref, bb_ref, Wc_ref, bc_ref,
                  Wd_ref, bd_ref, We_ref, be_ref, Wf_ref, bf_ref, S_ref,
                  xspv_ref, xspr_ref, adj_ref):
    f32 = jnp.float32
    x = jnp.concatenate([spv_ref[:], sh_ref[:], level_ref[:], spr_ref[:]],
                        axis=1)
    h1 = jnp.maximum(
        jnp.dot(x, Wa_ref[:], preferred_element_type=f32) + ba_ref[:], 0.0)
    nodef = jnp.dot(h1, Wb_ref[:], preferred_element_type=f32) + bb_ref[:]
    h = jnp.dot(nodef, Wc_ref[:], preferred_element_type=f32) + bc_ref[:]
    ml = jnp.dot(h, Wd_ref[:], preferred_element_type=f32) + bd_ref[:]
    mu = ml[:, :64]
    logstd = ml[:, 64:]
    z = mu + eps_ref[:] * jnp.exp(logstd)

    zt = jnp.concatenate([z, z, z, z], axis=1)
    chunks = [z[:, 16 * m:16 * (m + 1)] for m in range(4)]
    R = jnp.concatenate(
        [jnp.concatenate([c, c, c, c], axis=1) for c in chunks], axis=1)
    P = zt * R
    adj_ref[:] = jax.nn.sigmoid(
        jnp.dot(P, S_ref[:], preferred_element_type=f32))

    dh = jnp.maximum(
        jnp.dot(z, We_ref[:], preferred_element_type=f32) + be_ref[:], 0.0)
    out = jnp.dot(dh, Wf_ref[:], preferred_element_type=f32) + bf_ref[:]
    xspv_ref[:] = out[:, :64]
    xspr_ref[:] = out[:, 64:]


def kernel(dev_sh, dev_spv, dev_spr, dev_level, sh_W1, sh_b1, sh_W2, sh_b2,
           c1_W, c1_b, cmu_W, cmu_b, cls_W, cls_b, spv_W1, spv_b1, spv_W2,
           spv_b2, spr_W1, spr_b1, spr_W2, spr_b2, eps):
    B = dev_sh.shape[0]
    blk = 2048 if B % 2048 == 0 else B
    f32 = jnp.float32
    I4 = jnp.eye(4, dtype=f32)

    def bd(W):
        return jnp.kron(I4, W)

    def tb(b):
        return jnp.tile(b, 4)[None, :]

    Wa, ba = bd(sh_W1), tb(sh_b1)
    Wb, bb = bd(sh_W2), tb(sh_b2)
    Wc, bc = bd(c1_W), tb(c1_b)
    Wd = jnp.concatenate([bd(cmu_W), bd(cls_W)], axis=1)
    bdb = jnp.concatenate([tb(cmu_b), tb(cls_b)], axis=1)
    z32 = jnp.zeros((64, 32), f32)
    We = jnp.concatenate([bd(spv_W1), z32, bd(spr_W1), z32], axis=1)
    be = jnp.concatenate(
        [tb(spv_b1), jnp.zeros((1, 32), f32), tb(spr_b1),
         jnp.zeros((1, 32), f32)], axis=1)
    z3264 = jnp.zeros((32, 64), f32)
    Wf = jnp.concatenate([
        jnp.concatenate([bd(spv_W2), jnp.zeros((96, 64), f32)], axis=1),
        jnp.concatenate([z3264, z3264], axis=1),
        jnp.concatenate([jnp.zeros((96, 64), f32), bd(spr_W2)], axis=1),
        jnp.concatenate([z3264, z3264], axis=1)], axis=0)
    bf = jnp.concatenate([tb(spv_b2), tb(spr_b2)], axis=1)
    S = jnp.asarray(_S_NP)

    eps2 = eps.reshape(B, 64)

    row_spec = lambda w: pl.BlockSpec((blk, w), lambda i: (i, 0))
    full = lambda a: pl.BlockSpec(a.shape, lambda i: (0,) * a.ndim)

    xspv, xspr, adj = pl.pallas_call(
        _fused_kernel,
        grid=(B // blk,),
        in_specs=[row_spec(3), row_spec(3), row_spec(3), row_spec(3),
                  row_spec(64),
                  full(Wa), full(ba), full(Wb), full(bb), full(Wc), full(bc),
                  full(Wd), full(bdb), full(We), full(be), full(Wf), full(bf),
                  full(S)],
        out_specs=[row_spec(64), row_spec(64), row_spec(16)],
        out_shape=[jax.ShapeDtypeStruct((B, 64), f32),
                   jax.ShapeDtypeStruct((B, 64), f32),
                   jax.ShapeDtypeStruct((B, 16), f32)],
        compiler_params=pltpu.CompilerParams(
            dimension_semantics=("parallel",)),
    )(dev_spv, dev_sh, dev_level, dev_spr, eps2,
      Wa, ba, Wb, bb, Wc, bc, Wd, bdb, We, be, Wf, bf, S)

    return (xspv.reshape(B, 4, 16), xspr.reshape(B, 4, 16),
            adj.reshape(B, 4, 4))


# speedup vs baseline: 4.2378x; 1.5331x over previous
import numpy as np
import jax
import jax.numpy as jnp
from jax.experimental import pallas as pl
from jax.experimental.pallas import tpu as pltpu


def _gram_constants():
    Tt = np.zeros((64, 256), dtype=np.float32)
    Tr = np.zeros((64, 256), dtype=np.float32)
    S = np.zeros((256, 16), dtype=np.float32)
    for m in range(4):
        for n in range(4):
            for k in range(16):
                j = m * 64 + n * 16 + k
                Tt[n * 16 + k, j] = 1.0
                Tr[m * 16 + k, j] = 1.0
                S[j, n * 4 + m] = 1.0
    return Tt, Tr, S


_TT_NP, _TR_NP, _S_NP = _gram_constants()


def _fused_kernel(x_ref, eps_ref,
                  Wa_ref, ba_ref, Wb_ref, bb_ref, Wc_ref, bc_ref,
                  Wmu_ref, bmu_ref, Wls_ref, bls_ref,
                  We_ref, be_ref, Wf1_ref, bf1_ref, Wf2_ref, bf2_ref,
                  Tt_ref, Tr_ref, S_ref,
                  xspv_ref, xspr_ref, adj_ref):
    f32 = jnp.float32
    dot = lambda a, b: jnp.dot(a, b, preferred_element_type=f32)
    x = x_ref[:]
    h1 = jnp.maximum(dot(x, Wa_ref[:]) + ba_ref[:], 0.0)
    nodef = dot(h1, Wb_ref[:]) + bb_ref[:]
    h = dot(nodef, Wc_ref[:]) + bc_ref[:]
    mu = dot(h, Wmu_ref[:]) + bmu_ref[:]
    logstd = dot(h, Wls_ref[:]) + bls_ref[:]
    z = mu + eps_ref[:] * jnp.exp(logstd)

    P = dot(z, Tt_ref[:]) * dot(z, Tr_ref[:])
    adj_ref[:] = jax.nn.sigmoid(dot(P, S_ref[:]))

    dh = jnp.maximum(dot(z, We_ref[:]) + be_ref[:], 0.0)
    xspv_ref[:] = dot(dh, Wf1_ref[:]) + bf1_ref[:]
    xspr_ref[:] = dot(dh, Wf2_ref[:]) + bf2_ref[:]


def kernel(dev_sh, dev_spv, dev_spr, dev_level, sh_W1, sh_b1, sh_W2, sh_b2,
           c1_W, c1_b, cmu_W, cmu_b, cls_W, cls_b, spv_W1, spv_b1, spv_W2,
           spv_b2, spr_W1, spr_b1, spr_W2, spr_b2, eps):
    B = dev_sh.shape[0]
    blk = 2048 if B % 2048 == 0 else B
    f32 = jnp.float32
    I4 = jnp.eye(4, dtype=f32)

    def bd(W):
        return jnp.kron(I4, W)

    def tb(b):
        return jnp.tile(b, 4)[None, :]

    Wa, ba = bd(sh_W1), tb(sh_b1)
    Wb, bb = bd(sh_W2), tb(sh_b2)
    Wc, bc = bd(c1_W), tb(c1_b)
    Wmu, bmu = bd(cmu_W), tb(cmu_b)
    Wls, bls = bd(cls_W), tb(cls_b)
    z32 = jnp.zeros((64, 32), f32)
    We = jnp.concatenate([bd(spv_W1), z32, bd(spr_W1), z32], axis=1)
    be = jnp.concatenate(
        [tb(spv_b1), jnp.zeros((1, 32), f32), tb(spr_b1),
         jnp.zeros((1, 32), f32)], axis=1)
    zpad = jnp.zeros((32, 64), f32)
    Wf1 = jnp.concatenate(
        [bd(spv_W2), zpad, jnp.zeros((96, 64), f32), zpad], axis=0)
    Wf2 = jnp.concatenate(
        [jnp.zeros((96, 64), f32), zpad, bd(spr_W2), zpad], axis=0)
    bf1, bf2 = tb(spv_b2), tb(spr_b2)
    Tt, Tr, S = jnp.asarray(_TT_NP), jnp.asarray(_TR_NP), jnp.asarray(_S_NP)

    x12 = jnp.concatenate([dev_spv, dev_sh, dev_level, dev_spr], axis=1)
    eps2 = eps.reshape(B, 64)

    row_spec = lambda w: pl.BlockSpec((blk, w), lambda i: (i, 0))
    full = lambda a: pl.BlockSpec(a.shape, lambda i: (0,) * a.ndim)

    xspv, xspr, adj = pl.pallas_call(
        _fused_kernel,
        grid=(B // blk,),
        in_specs=[row_spec(12), row_spec(64),
                  full(Wa), full(ba), full(Wb), full(bb), full(Wc), full(bc),
                  full(Wmu), full(bmu), full(Wls), full(bls),
                  full(We), full(be), full(Wf1), full(bf1), full(Wf2),
                  full(bf2), full(Tt), full(Tr), full(S)],
        out_specs=[row_spec(64), row_spec(64), row_spec(16)],
        out_shape=[jax.ShapeDtypeStruct((B, 64), f32),
                   jax.ShapeDtypeStruct((B, 64), f32),
                   jax.ShapeDtypeStruct((B, 16), f32)],
        compiler_params=pltpu.CompilerParams(
            dimension_semantics=("parallel",)),
    )(x12, eps2,
      Wa, ba, Wb, bb, Wc, bc, Wmu, bmu, Wls, bls,
      We, be, Wf1, bf1, Wf2, bf2, Tt, Tr, S)

    return (xspv.reshape(B, 4, 16), xspr.reshape(B, 4, 16),
            adj.reshape(B, 4, 4))


# speedup vs baseline: 4.3312x; 1.0220x over previous
import numpy as np
import jax
import jax.numpy as jnp
from jax.experimental import pallas as pl
from jax.experimental.pallas import tpu as pltpu


def _gram_constants():
    Tt = np.zeros((64, 256), dtype=np.float32)
    Tr = np.zeros((64, 256), dtype=np.float32)
    S = np.zeros((256, 16), dtype=np.float32)
    for m in range(4):
        for n in range(4):
            for k in range(16):
                j = m * 64 + n * 16 + k
                Tt[n * 16 + k, j] = 1.0
                Tr[m * 16 + k, j] = 1.0
                S[j, n * 4 + m] = 1.0
    return Tt, Tr, S


_TT_NP, _TR_NP, _S_NP = _gram_constants()


def _fused_kernel(x_ref, eps_ref,
                  Wa_ref, ba_ref, Wb_ref, bb_ref, Wc_ref, bc_ref,
                  Wmu_ref, bmu_ref, Wls_ref, bls_ref,
                  We_ref, be_ref, Wf1_ref, bf1_ref, Wf2_ref, bf2_ref,
                  Tt_ref, Tr_ref, S_ref,
                  xspv_ref, xspr_ref, adj_ref):
    f32 = jnp.float32
    dot = lambda a, b: jnp.dot(a, b, preferred_element_type=f32)
    x = x_ref[:]
    h1 = jnp.maximum(dot(x, Wa_ref[:]) + ba_ref[:], 0.0)
    nodef = dot(h1, Wb_ref[:]) + bb_ref[:]
    h = dot(nodef, Wc_ref[:]) + bc_ref[:]
    mu = dot(h, Wmu_ref[:]) + bmu_ref[:]
    logstd = dot(h, Wls_ref[:]) + bls_ref[:]
    z = mu + eps_ref[:] * jnp.exp(logstd)

    P = dot(z, Tt_ref[:]) * dot(z, Tr_ref[:])
    adj_ref[:] = jax.nn.sigmoid(dot(P, S_ref[:]))

    dh = jnp.maximum(dot(z, We_ref[:]) + be_ref[:], 0.0)
    xspv_ref[:] = dot(dh, Wf1_ref[:]) + bf1_ref[:]
    xspr_ref[:] = dot(dh, Wf2_ref[:]) + bf2_ref[:]


def kernel(dev_sh, dev_spv, dev_spr, dev_level, sh_W1, sh_b1, sh_W2, sh_b2,
           c1_W, c1_b, cmu_W, cmu_b, cls_W, cls_b, spv_W1, spv_b1, spv_W2,
           spv_b2, spr_W1, spr_b1, spr_W2, spr_b2, eps):
    B = dev_sh.shape[0]
    blk = 8192 if B % 8192 == 0 else B
    f32 = jnp.float32
    I4 = jnp.eye(4, dtype=f32)

    def bd(W):
        return jnp.kron(I4, W)

    def tb(b):
        return jnp.tile(b, 4)[None, :]

    Wa, ba = bd(sh_W1), tb(sh_b1)
    Wb, bb = bd(sh_W2), tb(sh_b2)
    Wc, bc = bd(c1_W), tb(c1_b)
    Wmu, bmu = bd(cmu_W), tb(cmu_b)
    Wls, bls = bd(cls_W), tb(cls_b)
    z32 = jnp.zeros((64, 32), f32)
    We = jnp.concatenate([bd(spv_W1), z32, bd(spr_W1), z32], axis=1)
    be = jnp.concatenate(
        [tb(spv_b1), jnp.zeros((1, 32), f32), tb(spr_b1),
         jnp.zeros((1, 32), f32)], axis=1)
    zpad = jnp.zeros((32, 64), f32)
    Wf1 = jnp.concatenate(
        [bd(spv_W2), zpad, jnp.zeros((96, 64), f32), zpad], axis=0)
    Wf2 = jnp.concatenate(
        [jnp.zeros((96, 64), f32), zpad, bd(spr_W2), zpad], axis=0)
    bf1, bf2 = tb(spv_b2), tb(spr_b2)
    Tt, Tr, S = jnp.asarray(_TT_NP), jnp.asarray(_TR_NP), jnp.asarray(_S_NP)

    x12 = jnp.concatenate([dev_spv, dev_sh, dev_level, dev_spr], axis=1)
    eps2 = eps.reshape(B, 64)

    row_spec = lambda w: pl.BlockSpec((blk, w), lambda i: (i, 0))
    full = lambda a: pl.BlockSpec(a.shape, lambda i: (0,) * a.ndim)

    xspv, xspr, adj = pl.pallas_call(
        _fused_kernel,
        grid=(B // blk,),
        in_specs=[row_spec(12), row_spec(64),
                  full(Wa), full(ba), full(Wb), full(bb), full(Wc), full(bc),
                  full(Wmu), full(bmu), full(Wls), full(bls),
                  full(We), full(be), full(Wf1), full(bf1), full(Wf2),
                  full(bf2), full(Tt), full(Tr), full(S)],
        out_specs=[row_spec(64), row_spec(64), row_spec(16)],
        out_shape=[jax.ShapeDtypeStruct((B, 64), f32),
                   jax.ShapeDtypeStruct((B, 64), f32),
                   jax.ShapeDtypeStruct((B, 16), f32)],
        compiler_params=pltpu.CompilerParams(
            dimension_semantics=("parallel",)),
    )(x12, eps2,
      Wa, ba, Wb, bb, Wc, bc, Wmu, bmu, Wls, bls,
      We, be, Wf1, bf1, Wf2, bf2, Tt, Tr, S)

    return (xspv.reshape(B, 4, 16), xspr.reshape(B, 4, 16),
            adj.reshape(B, 4, 4))
